# baseline (speedup 1.0000x reference)
import jax
import jax.numpy as jnp
from jax import lax
from jax.experimental import pallas as pl
from jax.experimental.pallas import tpu as pltpu
from jax.experimental.pallas import tpu_sc as plsc

B = 16384
NUM_ENT = 100000
NUM_REL = 1000
S_DIM = 32
R_DIM = 64
GW = 128
L = 16
NC = 2
NS = 16
NW = NC * NS
PER_W = B // NW
C = 32
NCHUNK = PER_W // C
NG = C // L
BR = 1000

_S1 = -1.6666654611e-1
_S2 = 8.3321608736e-3
_S3 = -1.9515295891e-4
_PI = 3.14159265358979323846
_INV_PI = 1.0 / _PI
_MAGIC = 12582912.0


def _sin_poly(r):
    r2 = r * r
    return r + r * r2 * (_S1 + r2 * (_S2 + r2 * _S3))


def _sin_reduced(x):
    n_f = (x * _INV_PI + _MAGIC) - _MAGIC
    r = x - n_f * _PI
    s = _sin_poly(r)
    odd = (n_f.astype(jnp.int32) & 1) == 1
    return jnp.where(odd, -s, s)


def _repack_tc_body(*refs):
    for g in range(5):
        refs[20 + g][...] = jnp.concatenate(
            [refs[4 * g + t][...] for t in range(4)], axis=1)


def _repack_rel_body(rf, ri, out):
    out[...] = jnp.concatenate([rf[...], ri[...]], axis=1)


def _sc_body(*refs):
    samples = refs[0]
    packed = refs[1:6]
    relp = refs[6]
    out_hbm = refs[7]
    samp_v = refs[8:10]
    hidx = refs[10:12]
    ridx = refs[12:14]
    tidx = refs[14:16]
    out_v = refs[16]
    bufsets = (refs[17:28], refs[28:39])
    sems = refs[39:41]

    wid = lax.axis_index("s") * NC + lax.axis_index("c")
    lane = lax.iota(jnp.int32, L)

    def load_chunk(c, b):
        base = wid * PER_W + c * C
        pltpu.sync_copy(samples.at[pl.ds(base, C)], samp_v[b])

        def idx_body(g, carry):
            ids = g * L + lane
            for col, dst in ((0, hidx[b]), (1, ridx[b]), (2, tidx[b])):
                cv = jnp.full((L,), col, jnp.int32)
                v = plsc.load_gather(samp_v[b], [ids, cv]).astype(jnp.int32)
                dst[pl.ds(g * L, L)] = v
            return carry
        lax.fori_loop(0, NG, idx_body, 0)

        bufs = bufsets[b]
        for g in range(5):
            pltpu.async_copy(packed[g].at[hidx[b]], bufs[2 * g], sems[b])
            pltpu.async_copy(packed[g].at[tidx[b]], bufs[2 * g + 1], sems[b])
        pltpu.async_copy(relp.at[ridx[b]], bufs[10], sems[b])

    def wait_set(b):
        bufs = bufsets[b]
        for g in range(5):
            pltpu.make_async_copy(
                packed[g].at[pl.ds(0, C)], bufs[2 * g], sems[b]).wait()
            pltpu.make_async_copy(
                packed[g].at[pl.ds(0, C)], bufs[2 * g + 1], sems[b]).wait()
        pltpu.make_async_copy(
            relp.at[pl.ds(0, C)], bufs[10], sems[b]).wait()

    def compute_chunk(c, b):
        base = wid * PER_W + c * C
        bufs = bufsets[b]
        sv = samp_v[b]

        @plsc.parallel_loop(0, C, step=1, unroll=4)
        def sample_body(i):
            iv = jnp.full((L,), i, jnp.int32)
            year = plsc.load_gather(sv, [iv, jnp.full((L,), 3, jnp.int32)])
            month = plsc.load_gather(sv, [iv, jnp.full((L,), 4, jnp.int32)])
            day = plsc.load_gather(sv, [iv, jnp.full((L,), 5, jnp.int32)])

            acc = jnp.zeros((L,), jnp.float32)
            for q in range(2):
                qo = q * L

                def tb(pos, side):
                    buf = bufs[2 * (pos // 4) + side]
                    return buf[i, pl.ds(S_DIM * (pos % 4) + qo, L)]

                def te(s, side):
                    e = tb(18 + s, side) * _sin_reduced(
                        tb(6 + s, side) * year + tb(12 + s, side))
                    e = e + tb(14 + s, side) * _sin_poly(
                        tb(2 + s, side) * month + tb(8 + s, side))
                    e = e + tb(16 + s, side) * _sin_poly(
                        tb(4 + s, side) * day + tb(10 + s, side))
                    return e

                r1s = bufs[10][i, pl.ds(qo, L)]
                r1t = bufs[10][i, pl.ds(S_DIM + qo, L)]
                r2s = bufs[10][i, pl.ds(R_DIM + qo, L)]
                r2t = bufs[10][i, pl.ds(R_DIM + S_DIM + qo, L)]

                acc = acc + tb(0, 0) * r1s * tb(1, 1)
                acc = acc + tb(0, 1) * r2s * tb(1, 0)
                acc = acc + te(0, 0) * r1t * te(1, 1)
                acc = acc + te(0, 1) * r2t * te(1, 0)

            score = jnp.sum(acc) * 0.5
            plsc.store_scatter(out_v, [iv], jnp.full((L,), score),
                               mask=lane == 0)

        pltpu.sync_copy(out_v, out_hbm.at[pl.ds(base, C)])

    load_chunk(0, 0)

    def pair_body(j, carry):
        c0 = j * 2
        load_chunk(c0 + 1, 1)
        wait_set(0)
        compute_chunk(c0, 0)

        @pl.when(j < NCHUNK // 2 - 1)
        def _():
            load_chunk(c0 + 2, 0)

        wait_set(1)
        compute_chunk(c0 + 1, 1)
        return carry

    lax.fori_loop(0, NCHUNK // 2, pair_body, 0)


def kernel(samples, ent_embs_h, ent_embs_t, rel_embs_f, rel_embs_i,
           m_freq_h, m_freq_t, d_freq_h, d_freq_t, y_freq_h, y_freq_t,
           m_phi_h, m_phi_t, d_phi_h, d_phi_t, y_phi_h, y_phi_t,
           m_amps_h, m_amps_t, d_amps_h, d_amps_t, y_amps_h, y_amps_t):
    tables = (
        ent_embs_h, ent_embs_t, m_freq_h, m_freq_t,
        d_freq_h, d_freq_t, y_freq_h, y_freq_t,
        m_phi_h, m_phi_t, d_phi_h, d_phi_t,
        y_phi_h, y_phi_t, m_amps_h, m_amps_t,
        d_amps_h, d_amps_t, y_amps_h, y_amps_t,
    )

    in_spec = pl.BlockSpec((BR, S_DIM), lambda i: (i, 0))
    out_spec = pl.BlockSpec((BR, GW), lambda i: (i, 0))
    packed = pl.pallas_call(
        _repack_tc_body,
        grid=(NUM_ENT // BR,),
        in_specs=[in_spec] * 20,
        out_specs=[out_spec] * 5,
        out_shape=[jax.ShapeDtypeStruct((NUM_ENT, GW), jnp.float32)] * 5,
    )(*tables)

    relp = pl.pallas_call(
        _repack_rel_body,
        out_shape=jax.ShapeDtypeStruct((NUM_REL, 2 * R_DIM), jnp.float32),
    )(rel_embs_f, rel_embs_i)

    mesh = plsc.VectorSubcoreMesh(core_axis_name="c", subcore_axis_name="s")
    scratch = (
        [pltpu.VMEM((C, 6), jnp.float32)] * 2
        + [pltpu.VMEM((C,), jnp.int32)] * 6
        + [pltpu.VMEM((C,), jnp.float32)]
        + [pltpu.VMEM((C, GW), jnp.float32)] * 22
        + [pltpu.SemaphoreType.DMA] * 2
    )
    score = pl.kernel(
        _sc_body,
        mesh=mesh,
        out_type=jax.ShapeDtypeStruct((B,), jnp.float32),
        scratch_types=scratch,
        compiler_params=pltpu.CompilerParams(
            needs_layout_passes=False, use_tc_tiling_on_sc=False),
    )
    return score(samples, *packed, relp)


# speedup vs baseline: 1.1907x; 1.1907x over previous
import jax
import jax.numpy as jnp
from jax import lax
from jax.experimental import pallas as pl
from jax.experimental.pallas import tpu as pltpu
from jax.experimental.pallas import tpu_sc as plsc

B = 16384
S_DIM = 32
T_DIM = 32
R_DIM = 64
L = 16
NC = 2
NS = 16
NW = NC * NS
PER_W = B // NW
C = 32
NCHUNK = PER_W // C
NG = C // L

_S1 = -1.6666654611e-1
_S2 = 8.3321608736e-3
_S3 = -1.9515295891e-4
_PI = 3.14159265358979323846
_INV_PI = 1.0 / _PI
_MAGIC = 12582912.0


def _sin_poly(r):
    r2 = r * r
    return r + r * r2 * (_S1 + r2 * (_S2 + r2 * _S3))


def _sin_reduced(x):
    n_f = (x * _INV_PI + _MAGIC) - _MAGIC
    r = x - n_f * _PI
    s = _sin_poly(r)
    odd = (n_f.astype(jnp.int32) & 1) == 1
    return jnp.where(odd, -s, s)


def _sc_body(*refs):
    samples = refs[0]
    tables = refs[1:3] + refs[5:23]
    rel_f = refs[3]
    rel_i = refs[4]
    out_hbm = refs[23]
    samp_v = refs[24:26]
    hidx = refs[26:28]
    ridx = refs[28:30]
    tidx = refs[30:32]
    out_v = refs[32]
    bufsets = (refs[33:75], refs[75:117])
    sems = refs[117:119]


    wid = lax.axis_index("s") * NC + lax.axis_index("c")
    lane = lax.iota(jnp.int32, L)

    def load_chunk(c, b):
        base = wid * PER_W + c * C
        pltpu.sync_copy(samples.at[pl.ds(base, C)], samp_v[b])

        def idx_body(g, carry):
            ids = g * L + lane
            for col, dst in ((0, hidx[b]), (1, ridx[b]), (2, tidx[b])):
                cv = jnp.full((L,), col, jnp.int32)
                v = plsc.load_gather(samp_v[b], [ids, cv]).astype(jnp.int32)
                dst[pl.ds(g * L, L)] = v
            return carry
        lax.fori_loop(0, NG, idx_body, 0)

        bufs = bufsets[b]
        for k, tbl in enumerate(tables):
            pltpu.async_copy(tbl.at[hidx[b]], bufs[2 * k], sems[b])
            pltpu.async_copy(tbl.at[tidx[b]], bufs[2 * k + 1], sems[b])
        pltpu.async_copy(rel_f.at[ridx[b]], bufs[40], sems[b])
        pltpu.async_copy(rel_i.at[ridx[b]], bufs[41], sems[b])

    def wait_set(b):
        bufs = bufsets[b]
        for k, tbl in enumerate(tables):
            pltpu.make_async_copy(
                tbl.at[pl.ds(0, C)], bufs[2 * k], sems[b]).wait()
            pltpu.make_async_copy(
                tbl.at[pl.ds(0, C)], bufs[2 * k + 1], sems[b]).wait()
        pltpu.make_async_copy(rel_f.at[pl.ds(0, C)], bufs[40], sems[b]).wait()
        pltpu.make_async_copy(rel_i.at[pl.ds(0, C)], bufs[41], sems[b]).wait()

    def compute_chunk(c, b):
        base = wid * PER_W + c * C
        bufs = bufsets[b]
        sv = samp_v[b]

        @plsc.parallel_loop(0, C, step=1, unroll=4)
        def sample_body(i):
            iv = jnp.full((L,), i, jnp.int32)
            year = plsc.load_gather(sv, [iv, jnp.full((L,), 3, jnp.int32)])
            month = plsc.load_gather(sv, [iv, jnp.full((L,), 4, jnp.int32)])
            day = plsc.load_gather(sv, [iv, jnp.full((L,), 5, jnp.int32)])

            acc = jnp.zeros((L,), jnp.float32)
            for q in range(2):
                qo = q * L

                def tb(pos, side):
                    return bufs[2 * pos + side][i, pl.ds(qo, L)]

                def te(s, side):
                    e = tb(18 + s, side) * _sin_reduced(
                        tb(6 + s, side) * year + tb(12 + s, side))
                    e = e + tb(14 + s, side) * _sin_poly(
                        tb(2 + s, side) * month + tb(8 + s, side))
                    e = e + tb(16 + s, side) * _sin_poly(
                        tb(4 + s, side) * day + tb(10 + s, side))
                    return e

                r1s = bufs[40][i, pl.ds(qo, L)]
                r1t = bufs[40][i, pl.ds(S_DIM + qo, L)]
                r2s = bufs[41][i, pl.ds(qo, L)]
                r2t = bufs[41][i, pl.ds(S_DIM + qo, L)]

                acc = acc + tb(0, 0) * r1s * tb(1, 1)
                acc = acc + tb(0, 1) * r2s * tb(1, 0)
                acc = acc + te(0, 0) * r1t * te(1, 1)
                acc = acc + te(0, 1) * r2t * te(1, 0)

            score = jnp.sum(acc) * 0.5
            plsc.store_scatter(out_v, [iv], jnp.full((L,), score),
                               mask=lane == 0)

        pltpu.sync_copy(out_v, out_hbm.at[pl.ds(base, C)])

    load_chunk(0, 0)

    def pair_body(j, carry):
        c0 = j * 2
        load_chunk(c0 + 1, 1)
        wait_set(0)
        compute_chunk(c0, 0)

        @pl.when(j < NCHUNK // 2 - 1)
        def _():
            load_chunk(c0 + 2, 0)

        wait_set(1)
        compute_chunk(c0 + 1, 1)
        return carry

    lax.fori_loop(0, NCHUNK // 2, pair_body, 0)


def kernel(samples, ent_embs_h, ent_embs_t, rel_embs_f, rel_embs_i,
           m_freq_h, m_freq_t, d_freq_h, d_freq_t, y_freq_h, y_freq_t,
           m_phi_h, m_phi_t, d_phi_h, d_phi_t, y_phi_h, y_phi_t,
           m_amps_h, m_amps_t, d_amps_h, d_amps_t, y_amps_h, y_amps_t):
    mesh = plsc.VectorSubcoreMesh(core_axis_name="c", subcore_axis_name="s")
    bufset = (
        [pltpu.VMEM((C, S_DIM), jnp.float32)] * 40
        + [pltpu.VMEM((C, R_DIM), jnp.float32)] * 2
    )
    scratch = (
        [pltpu.VMEM((C, 6), jnp.float32)] * 2
        + [pltpu.VMEM((C,), jnp.int32)] * 6
        + [pltpu.VMEM((C,), jnp.float32)]
        + bufset * 2
        + [pltpu.SemaphoreType.DMA] * 2
    )
    run = pl.kernel(
        _sc_body,
        mesh=mesh,
        out_type=jax.ShapeDtypeStruct((B,), jnp.float32),
        scratch_types=scratch,
        compiler_params=pltpu.CompilerParams(
            needs_layout_passes=False, use_tc_tiling_on_sc=False),
    )
    return run(samples, ent_embs_h, ent_embs_t, rel_embs_f, rel_embs_i,
               m_freq_h, m_freq_t, d_freq_h, d_freq_t, y_freq_h, y_freq_t,
               m_phi_h, m_phi_t, d_phi_h, d_phi_t, y_phi_h, y_phi_t,
               m_amps_h, m_amps_t, d_amps_h, d_amps_t, y_amps_h, y_amps_t)
